# baseline (speedup 1.0000x reference)
import functools

import jax
import jax.numpy as jnp
from jax import lax
from jax.experimental import pallas as pl
from jax.experimental.pallas import tpu as pltpu
from jax.experimental.pallas import tpu_sc as plsc

SEQ_LEN = 2048
BT_SIZE = 4
D_MODEL = 4096
ROWS = SEQ_LEN * BT_SIZE

NC = 2
NS = 16
NW = NC * NS
RPW = ROWS // NW
C = 8
NCHUNK = RPW // C
NBUF = 3
L = 16


def _sc_body(x_hbm, p_hbm, o_hbm, pos_v, bufs, in_sems, out_sems):
    wid = lax.axis_index("s") * NC + lax.axis_index("c")
    base = wid * RPW
    pstart = lax.rem(base, SEQ_LEN)
    pltpu.sync_copy(p_hbm.at[pl.ds(pstart, RPW)], pos_v.at[pl.ds(0, RPW)])

    row_idx = lax.iota(jnp.int32, L)
    col_idx = jnp.full((L,), D_MODEL, jnp.int32)
    col_mask = row_idx < C

    def start_in(k, s):
        pltpu.make_async_copy(
            x_hbm.at[pl.ds(base + k * C, C), :],
            bufs.at[s, :, pl.ds(0, D_MODEL)],
            in_sems.at[s],
        ).start()

    def wait_in(k, s):
        pltpu.make_async_copy(
            x_hbm.at[pl.ds(base + k * C, C), :],
            bufs.at[s, :, pl.ds(0, D_MODEL)],
            in_sems.at[s],
        ).wait()

    def put_col(k, s):
        vals = pos_v[pl.ds(k * C, L)]
        plsc.store_scatter(bufs.at[s], [row_idx, col_idx], vals, mask=col_mask)

    def out_copy(k, s):
        return pltpu.make_async_copy(
            bufs.at[s],
            o_hbm.at[pl.ds(base + k * C, C), :],
            out_sems.at[s],
        )

    for s in range(NBUF):
        start_in(s, s)

    groups = [
        list(range(k0, min(k0 + NBUF, NCHUNK))) for k0 in range(0, NCHUNK, NBUF)
    ]
    for grp in groups:
        for k in grp:
            b = k % NBUF
            put_col(k, b)
            wait_in(k, b)
            out_copy(k, b).start()
        for k in grp:
            b = k % NBUF
            out_copy(k, b).wait()
            if k + NBUF < NCHUNK:
                start_in(k + NBUF, b)


def kernel(inputs, pos_table):
    x = inputs.reshape(ROWS, D_MODEL)
    p = pos_table.reshape(SEQ_LEN)
    mesh = plsc.VectorSubcoreMesh(core_axis_name="c", subcore_axis_name="s")
    sc_copy = functools.partial(
        pl.kernel,
        mesh=mesh,
        out_type=jax.ShapeDtypeStruct((ROWS, D_MODEL + 1), jnp.float32),
        scratch_types=[
            pltpu.VMEM((RPW + L,), jnp.float32),
            pltpu.VMEM((NBUF, C, D_MODEL + 1), jnp.float32),
            pltpu.SemaphoreType.DMA((NBUF,)),
            pltpu.SemaphoreType.DMA((NBUF,)),
        ],
        compiler_params=pltpu.CompilerParams(needs_layout_passes=False),
    )(_sc_body)
    out = sc_copy(x, p)
    return out.reshape(BT_SIZE, SEQ_LEN, D_MODEL + 1)
